# Initial kernel scaffold; baseline (speedup 1.0000x reference)
import jax
import jax.numpy as jnp
from jax.experimental import pallas as pl


def kernel(x, weight):
    raise NotImplementedError("write your pallas kernel here")


# speedup vs baseline: 977.4283x; 977.4283x over previous
import jax
import jax.numpy as jnp
from jax.experimental import pallas as pl
from jax.experimental.pallas import tpu as pltpu


def _conv_body(w_ref, x_ref, out_ref):
    x = x_ref[0]
    xp = jnp.pad(x, ((1, 1), (1, 1), (1, 3)))
    xf = xp.reshape(66 * 66, 68)

    k_i = jax.lax.broadcasted_iota(jnp.int32, (68, 96), 0)
    c_i = jax.lax.broadcasted_iota(jnp.int32, (68, 96), 1)
    sel = (k_i == 4 * (c_i % 16) + c_i // 16).astype(jnp.float32)
    z = jax.lax.dot_general(
        xf, sel, (((1,), (0,)), ((), ())), preferred_element_type=jnp.float32
    )
    z3 = z.reshape(66, 66, 96)
    yf = [z3[:, :, m * 16:(m + 1) * 16].reshape(66, 66 * 16) for m in range(6)]

    acc = [None] * 8
    for m in range(6):
        pairs = [(cp, m - cp) for cp in range(4) if 0 <= m - cp <= 2]
        for di in range(3):
            for dj in range(3):
                slab = yf[m][di:di + 64, dj * 16:dj * 16 + 1024]
                for o in range(8):
                    wsc = None
                    for (cp, dk) in pairs:
                        wv = w_ref[o, cp * 27 + di * 9 + dj * 3 + dk]
                        wsc = wv if wsc is None else wsc + wv
                    term = wsc * slab
                    acc[o] = term if acc[o] is None else acc[o] + term

    for o in range(8):
        out_ref[o, 0, :, :] = acc[o]


def kernel(x, weight):
    xr = x.reshape(4, 64, 64, 64)
    wr = weight.reshape(8, 108)
    out = pl.pallas_call(
        _conv_body,
        grid=(4,),
        in_specs=[
            pl.BlockSpec(memory_space=pltpu.SMEM),
            pl.BlockSpec((1, 64, 64, 64), lambda c: (c, 0, 0, 0)),
        ],
        out_specs=pl.BlockSpec((8, 1, 64, 1024), lambda c: (0, c, 0, 0)),
        out_shape=jax.ShapeDtypeStruct((8, 4, 64, 1024), jnp.float32),
        compiler_params=pltpu.CompilerParams(vmem_limit_bytes=100 * 1024 * 1024),
    )(wr, xr)
    return out.reshape(8, 64, 64, 64)[None]
